# baseline (speedup 1.0000x reference)
import jax
import jax.numpy as jnp
from jax.experimental import pallas as pl
from jax.experimental.pallas import tpu as pltpu

_USER_CHUNKS = 8
_ITEM_CHUNKS = 1
_USER_ROWS = 500_000
_ITEM_ROWS = 50_000


def _copy_body(u_ref, i_ref, ou_ref, oi_ref, sems):
    copies = []
    chunk = _USER_ROWS // _USER_CHUNKS
    for c in range(_USER_CHUNKS):
        sl = pl.ds(c * chunk, chunk)
        copies.append(pltpu.make_async_copy(
            u_ref.at[sl, :], ou_ref.at[sl, :], sems.at[c]))
    chunk = _ITEM_ROWS // _ITEM_CHUNKS
    for c in range(_ITEM_CHUNKS):
        sl = pl.ds(c * chunk, chunk)
        copies.append(pltpu.make_async_copy(
            i_ref.at[sl, :], oi_ref.at[sl, :], sems.at[_USER_CHUNKS + c]))
    for cp in copies:
        cp.start()
    for cp in copies:
        cp.wait()


def kernel(emb_user, emb_item):
    u = emb_user.reshape(_USER_ROWS, 128)
    it = emb_item.reshape(_ITEM_ROWS, 128)
    hbm = pl.BlockSpec(memory_space=pltpu.MemorySpace.HBM)
    out_u, out_it = pl.pallas_call(
        _copy_body,
        in_specs=[hbm, hbm],
        out_specs=[hbm, hbm],
        out_shape=[
            jax.ShapeDtypeStruct(u.shape, u.dtype),
            jax.ShapeDtypeStruct(it.shape, it.dtype),
        ],
        scratch_shapes=[pltpu.SemaphoreType.DMA((_USER_CHUNKS + _ITEM_CHUNKS,))],
    )(u, it)
    return (out_u.reshape(emb_user.shape), out_it.reshape(emb_item.shape))


# speedup vs baseline: 6.5483x; 6.5483x over previous
import functools

import jax
import jax.numpy as jnp
from jax import lax
from jax.experimental import pallas as pl
from jax.experimental.pallas import tpu as pltpu
from jax.experimental.pallas import tpu_sc as plsc

_NC, _NS = 2, 16
_NW = _NC * _NS
_TOT_U = 1_000_000 * 64
_TOT_I = 100_000 * 64
_PER_U = _TOT_U // _NW
_PER_I = _TOT_I // _NW
_CH = 50_000
_USER_CHUNKS = _PER_U // _CH
_ITEM_CHUNKS = _PER_I // _CH

_mesh = plsc.VectorSubcoreMesh(core_axis_name="c", subcore_axis_name="s")


def _copy_shard(src, dst, base, n_chunks, bufs, in_sems, out_sems):
    def gather(g, b):
        return pltpu.make_async_copy(
            src.at[pl.ds(base + g * _CH, _CH)], bufs[b], in_sems.at[b])

    def scatter(g, b):
        return pltpu.make_async_copy(
            bufs[b], dst.at[pl.ds(base + g * _CH, _CH)], out_sems.at[b])

    gather(0, 0).start()
    for g in range(n_chunks):
        b = g % 2
        gather(g, b).wait()
        scatter(g, b).start()
        if g + 1 < n_chunks:
            if g >= 1:
                scatter(g - 1, 1 - b).wait()
            gather(g + 1, 1 - b).start()
    if n_chunks >= 2:
        scatter(n_chunks - 2, (n_chunks - 2) % 2).wait()
    scatter(n_chunks - 1, (n_chunks - 1) % 2).wait()


@functools.partial(
    pl.kernel,
    out_type=[
        jax.ShapeDtypeStruct((_TOT_U,), jnp.float32),
        jax.ShapeDtypeStruct((_TOT_I,), jnp.float32),
    ],
    mesh=_mesh,
    scratch_types=[
        pltpu.VMEM((_CH,), jnp.float32),
        pltpu.VMEM((_CH,), jnp.float32),
        pltpu.SemaphoreType.DMA((2,)),
        pltpu.SemaphoreType.DMA((2,)),
    ],
)
def _sc_copy(u_hbm, i_hbm, out_u, out_i, buf0, buf1, in_sems, out_sems):
    wid = lax.axis_index("s") * _NC + lax.axis_index("c")
    bufs = (buf0, buf1)
    _copy_shard(u_hbm, out_u, wid * _PER_U, _USER_CHUNKS, bufs, in_sems, out_sems)
    _copy_shard(i_hbm, out_i, wid * _PER_I, _ITEM_CHUNKS, bufs, in_sems, out_sems)


def kernel(emb_user, emb_item):
    out_u, out_i = _sc_copy(emb_user.reshape(_TOT_U), emb_item.reshape(_TOT_I))
    return (out_u.reshape(1_000_000, 64), out_i.reshape(100_000, 64))
